# baseline (speedup 1.0000x reference)
import jax
import jax.numpy as jnp
from jax.experimental import pallas as pl

_B = 128
_FEATURES = [256, 64, 32, 3]
_BRANCHES = [4, 8, 16]
_NL = [1, 4, 32, 512]
_N_COND = 1
_N_GLOBAL = 8


def _dot(Wt, X):
    return jax.lax.dot_general(Wt, X, (((1,), (0,)), ((), ())),
                               preferred_element_type=jnp.float32)


def _bcast(Xb, n):
    if n == 1:
        return Xb
    d = Xb.shape[0]
    return jnp.broadcast_to(Xb[:, None, :], (d, n, _B)).reshape(d, n * _B)


def _rep_nodes(X, s):
    d, m = X.shape
    n = m // _B
    X3 = X.reshape(d, n, 1, _B)
    return jnp.broadcast_to(X3, (d, n, s, _B)).reshape(d, n * s * _B)


def _body(random_vector, cond, W_hlv, b_hlv, W_br, W_red, W_amsg, W_aupd,
          W_cmsg, W_cupd, scale, out_ref):
    cond_t = cond[...].T
    X0 = jnp.concatenate([cond_t, random_vector[...].T[_N_COND:]], axis=0)
    X = [X0]

    for il in range(3):
        d_in, d_out, br = _FEATURES[il], _FEATURES[il + 1], _BRANCHES[il]
        n = _NL[il]
        Xl = X[il]
        if n == 1:
            pooled = Xl[:_FEATURES[-1]]
        else:
            pooled = Xl[:_FEATURES[-1]].reshape(_FEATURES[-1], n, _B).mean(axis=1)
        hlv_in = jnp.concatenate([pooled, cond_t], axis=0)
        glob = jax.nn.relu(_dot(W_hlv[il][...].T, hlv_in) + b_hlv[il][...].T)

        feats = jnp.concatenate([
            Xl, _bcast(cond_t, n), _bcast(glob, n),
        ], axis=0)
        ch = jax.nn.relu(_dot(W_br[il][...].T, feats))
        ch = (ch.reshape(br, d_out, n, _B).transpose(1, 2, 0, 3)
              .reshape(d_out, n * br * _B))

        Wred_t = W_red[il][...].T
        X = [_dot(Wred_t, Xk) for Xk in X]
        X.append(ch)
        L = il + 1
        nL = _NL[L]
        cond_L = _bcast(cond_t, nL)
        glob_L = _bcast(glob, nL)

        Wam_t = W_amsg[il][...].T
        agg = jnp.zeros((d_out, nL * _B), jnp.float32)
        for a in range(L):
            stride = 1
            for bb in _BRANCHES[a:L]:
                stride *= bb
            src = _rep_nodes(X[a], stride)
            ea = jnp.full((1, nL * _B), float(L - a), jnp.float32)
            F = jnp.concatenate([src, X[L], ea, cond_L, glob_L], axis=0)
            agg = agg + jax.nn.relu(_dot(Wam_t, F))

        Wau = W_aupd[il][...]
        Wau_t = Wau[:d_out].T
        Wau_full_t = Wau.T
        Xn = [jax.nn.relu(_dot(Wau_t, X[k])) for k in range(L)]
        Xn.append(jax.nn.relu(_dot(Wau_full_t,
                                   jnp.concatenate([X[L], agg], axis=0))))
        X = Xn

        Wcm_t = W_cmsg[il][...].T
        ng = nL // br
        XL4 = X[L].reshape(d_out, ng, br, _B)
        cagg = jnp.zeros((d_out, nL * _B), jnp.float32)
        for i in range(br):
            src_i = jnp.broadcast_to(XL4[:, :, i:i + 1, :],
                                     (d_out, ng, br, _B)).reshape(d_out, nL * _B)
            F = jnp.concatenate([src_i, X[L], cond_L, glob_L], axis=0)
            cagg = cagg + jax.nn.relu(_dot(Wcm_t, F))

        Wcu = W_cupd[il][...]
        Wcu_t = Wcu[:d_out].T
        Wcu_full_t = Wcu.T
        Xn = [jax.nn.relu(_dot(Wcu_t, X[k])) for k in range(L)]
        Xn.append(jax.nn.relu(_dot(Wcu_full_t,
                                   jnp.concatenate([X[L], cagg], axis=0))))
        X = Xn

    X3 = (X[3] * scale[...].T).reshape(_FEATURES[-1], _NL[3], _B)
    out_ref[...] = X3.transpose(0, 2, 1).reshape(_FEATURES[-1], _B * _NL[3])


def _pallas_body(rv, cond,
                 hlv0, bh0, br0, red0, am0, au0, cm0, cu0,
                 hlv1, bh1, br1, red1, am1, au1, cm1, cu1,
                 hlv2, bh2, br2, red2, am2, au2, cm2, cu2,
                 scale, out_ref):
    _body(rv, cond,
          [hlv0, hlv1, hlv2], [bh0, bh1, bh2], [br0, br1, br2],
          [red0, red1, red2], [am0, am1, am2], [au0, au1, au2],
          [cm0, cm1, cm2], [cu0, cu1, cu2], scale, out_ref)


def kernel(random_vector, cond,
           W_hlv_0, b_hlv_0, W_br_0, W_red_0, W_amsg_0, W_aupd_0, W_cmsg_0, W_cupd_0,
           W_hlv_1, b_hlv_1, W_br_1, W_red_1, W_amsg_1, W_aupd_1, W_cmsg_1, W_cupd_1,
           W_hlv_2, b_hlv_2, W_br_2, W_red_2, W_amsg_2, W_aupd_2, W_cmsg_2, W_cupd_2,
           scale):
    operands = (
        random_vector, cond,
        W_hlv_0, b_hlv_0.reshape(1, _N_GLOBAL), W_br_0, W_red_0, W_amsg_0,
        W_aupd_0, W_cmsg_0, W_cupd_0,
        W_hlv_1, b_hlv_1.reshape(1, _N_GLOBAL), W_br_1, W_red_1, W_amsg_1,
        W_aupd_1, W_cmsg_1, W_cupd_1,
        W_hlv_2, b_hlv_2.reshape(1, _N_GLOBAL), W_br_2, W_red_2, W_amsg_2,
        W_aupd_2, W_cmsg_2, W_cupd_2,
        scale.reshape(1, _FEATURES[-1]),
    )
    out = pl.pallas_call(
        _pallas_body,
        out_shape=jax.ShapeDtypeStruct((_FEATURES[-1], _B * _NL[3]), jnp.float32),
    )(*operands)
    return out.T


# speedup vs baseline: 1.0549x; 1.0549x over previous
import jax
import jax.numpy as jnp
from jax.experimental import pallas as pl

_B = 128
_FEATURES = [256, 64, 32, 3]
_BRANCHES = [4, 8, 16]
_NL = [1, 4, 32, 512]
_N_COND = 1
_N_GLOBAL = 8


def _dot(Wt, X):
    return jax.lax.dot_general(Wt, X, (((1,), (0,)), ((), ())),
                               preferred_element_type=jnp.float32)


def _bcast(Xb, n):
    if n == 1:
        return Xb
    d = Xb.shape[0]
    return jnp.broadcast_to(Xb[:, None, :], (d, n, _B)).reshape(d, n * _B)


def _rep_nodes(X, s):
    d, m = X.shape
    n = m // _B
    X3 = X.reshape(d, n, 1, _B)
    return jnp.broadcast_to(X3, (d, n, s, _B)).reshape(d, n * s * _B)


def _body(random_vector, cond, W_hlv, b_hlv, W_br, W_red, W_amsg, W_aupd,
          W_cmsg, W_cupd, scale, out_ref):
    cond_t = cond[...].T
    X0 = jnp.concatenate([cond_t, random_vector[...].T[_N_COND:]], axis=0)
    X = [X0]

    for il in range(3):
        d_in, d_out, br = _FEATURES[il], _FEATURES[il + 1], _BRANCHES[il]
        n = _NL[il]
        Xl = X[il]
        if n == 1:
            pooled = Xl[:_FEATURES[-1]]
        else:
            pooled = Xl[:_FEATURES[-1]].reshape(_FEATURES[-1], n, _B).mean(axis=1)
        hlv_in = jnp.concatenate([pooled, cond_t], axis=0)
        glob = jax.nn.relu(_dot(W_hlv[il][...].T, hlv_in) + b_hlv[il][...].T)

        feats = jnp.concatenate([
            Xl, _bcast(cond_t, n), _bcast(glob, n),
        ], axis=0)
        ch = jax.nn.relu(_dot(W_br[il][...].T, feats))
        ch = (ch.reshape(br, d_out, n, _B).transpose(1, 2, 0, 3)
              .reshape(d_out, n * br * _B))

        Wred_t = W_red[il][...].T
        X = [_dot(Wred_t, Xk) for Xk in X]
        X.append(ch)
        L = il + 1
        nL = _NL[L]
        cond_L = _bcast(cond_t, nL)
        glob_L = _bcast(glob, nL)

        Wam_t = W_amsg[il][...].T
        a_tail = jnp.concatenate([cond_L, glob_L], axis=0)
        agg = jnp.zeros((d_out, nL * _B), jnp.float32)
        for a in range(L):
            stride = 1
            for bb in _BRANCHES[a:L]:
                stride *= bb
            src = _rep_nodes(X[a], stride)
            ea = jnp.full((1, nL * _B), float(L - a), jnp.float32)
            F = jnp.concatenate([src, X[L], ea, a_tail], axis=0)
            agg = agg + jax.nn.relu(_dot(Wam_t, F))

        Wau = W_aupd[il][...]
        Wau_t = Wau[:d_out].T
        Wau_full_t = Wau.T
        Xn = [jax.nn.relu(_dot(Wau_t, X[k])) for k in range(L)]
        Xn.append(jax.nn.relu(_dot(Wau_full_t,
                                   jnp.concatenate([X[L], agg], axis=0))))
        X = Xn

        Wcm_t = W_cmsg[il][...].T
        ng = nL // br
        XL4 = X[L].reshape(d_out, ng, br, _B)
        c_tail = jnp.concatenate([X[L], cond_L, glob_L], axis=0)
        cagg = jnp.zeros((d_out, nL * _B), jnp.float32)
        for i in range(br):
            src_i = jnp.broadcast_to(XL4[:, :, i:i + 1, :],
                                     (d_out, ng, br, _B)).reshape(d_out, nL * _B)
            F = jnp.concatenate([src_i, c_tail], axis=0)
            cagg = cagg + jax.nn.relu(_dot(Wcm_t, F))

        Wcu = W_cupd[il][...]
        Wcu_t = Wcu[:d_out].T
        Wcu_full_t = Wcu.T
        Xn = [jax.nn.relu(_dot(Wcu_t, X[k])) for k in range(L)]
        Xn.append(jax.nn.relu(_dot(Wcu_full_t,
                                   jnp.concatenate([X[L], cagg], axis=0))))
        X = Xn

    X3 = (X[3] * scale[...].T).reshape(_FEATURES[-1], _NL[3], _B)
    out_ref[...] = X3.transpose(0, 2, 1).reshape(_FEATURES[-1], _B * _NL[3])


def _pallas_body(rv, cond,
                 hlv0, bh0, br0, red0, am0, au0, cm0, cu0,
                 hlv1, bh1, br1, red1, am1, au1, cm1, cu1,
                 hlv2, bh2, br2, red2, am2, au2, cm2, cu2,
                 scale, out_ref):
    _body(rv, cond,
          [hlv0, hlv1, hlv2], [bh0, bh1, bh2], [br0, br1, br2],
          [red0, red1, red2], [am0, am1, am2], [au0, au1, au2],
          [cm0, cm1, cm2], [cu0, cu1, cu2], scale, out_ref)


def kernel(random_vector, cond,
           W_hlv_0, b_hlv_0, W_br_0, W_red_0, W_amsg_0, W_aupd_0, W_cmsg_0, W_cupd_0,
           W_hlv_1, b_hlv_1, W_br_1, W_red_1, W_amsg_1, W_aupd_1, W_cmsg_1, W_cupd_1,
           W_hlv_2, b_hlv_2, W_br_2, W_red_2, W_amsg_2, W_aupd_2, W_cmsg_2, W_cupd_2,
           scale):
    operands = (
        random_vector, cond,
        W_hlv_0, b_hlv_0.reshape(1, _N_GLOBAL), W_br_0, W_red_0, W_amsg_0,
        W_aupd_0, W_cmsg_0, W_cupd_0,
        W_hlv_1, b_hlv_1.reshape(1, _N_GLOBAL), W_br_1, W_red_1, W_amsg_1,
        W_aupd_1, W_cmsg_1, W_cupd_1,
        W_hlv_2, b_hlv_2.reshape(1, _N_GLOBAL), W_br_2, W_red_2, W_amsg_2,
        W_aupd_2, W_cmsg_2, W_cupd_2,
        scale.reshape(1, _FEATURES[-1]),
    )
    out = pl.pallas_call(
        _pallas_body,
        out_shape=jax.ShapeDtypeStruct((_FEATURES[-1], _B * _NL[3]), jnp.float32),
    )(*operands)
    return out.T
